# baseline (speedup 1.0000x reference)
import jax
import jax.numpy as jnp
from jax.experimental import pallas as pl
from jax.experimental.pallas import tpu as pltpu

_EPS2 = 400.0
_MIN_SAMPLES = 5.0
_ITERS = 24
_T = 256


def _dbscan_body(x_ref, out_ref, mask_ref, sqr, denr, denc, lr, lc,
                 nmc, l1c, jc, rankc, ccr, ccc, bminc, done):
    N = x_ref.shape[0]
    T = _T if N % _T == 0 else N
    NT = N // T
    BIG = jnp.float32(N)
    f32 = jnp.float32
    hp = jax.lax.Precision.HIGHEST

    def loop(body):
        jax.lax.fori_loop(0, NT, lambda i, c: (body(i), 0)[1], 0)

    x = x_ref[...]
    sqr[...] = jnp.sum(x * x, axis=1, keepdims=True)
    denc[...] = jnp.zeros_like(denc[...])
    ones_t = jnp.ones((T, 1), f32)

    def p1(rt):
        sl = pl.ds(rt * T, T)
        xr = x_ref[sl, :]
        mm = jax.lax.dot_general(xr, x, (((1,), (1,)), ((), ())),
                                 precision=jax.lax.Precision.DEFAULT)
        sqc_b = jax.lax.dot_general(ones_t, sqr[...], (((1,), (1,)), ((), ())),
                                    precision=hp)
        d2 = (sqr[sl, :] + sqc_b) - 2.0 * mm
        nb = jnp.maximum(d2, 0.0) <= _EPS2
        mask_ref[sl, :] = nb.astype(jnp.int8)
        nbf = nb.astype(f32)
        denc[...] = denc[...] + jnp.sum(nbf, axis=0, keepdims=True)
        denr[sl, :] = jnp.sum(nbf, axis=1, keepdims=True)
    loop(p1)

    def p2(rt):
        sl = pl.ds(rt * T, T)
        m32 = mask_ref[sl, :].astype(jnp.int32)
        corer_t = (denr[sl, :] >= _MIN_SAMPLES).astype(jnp.int32)
        corec_i = (denc[...] >= _MIN_SAMPLES).astype(jnp.int32)
        mask_ref[sl, :] = (m32 * (1 + 2 * corer_t + 4 * corec_i)).astype(jnp.int8)
    loop(p2)

    iota_r = jax.lax.broadcasted_iota(jnp.int32, (N, 1), 0).astype(f32)
    iota_c = jax.lax.broadcasted_iota(jnp.int32, (1, N), 1).astype(f32)
    lr[...] = iota_r
    lc[...] = iota_c
    done[0] = 0

    iota_tr = jax.lax.broadcasted_iota(jnp.int32, (T, 1), 0)

    def transpose_cols_to_rows(src_c, dst_r):
        def tbody(rt):
            sl = pl.ds(rt * T, T)
            rowg = (iota_tr + rt * T).astype(f32)
            eq = (iota_c == rowg).astype(f32)
            dst_r[sl, :] = jax.lax.dot_general(
                eq, src_c[...], (((1,), (1,)), ((), ())), precision=hp)
        loop(tbody)

    def iter_body(_, carry):
        @pl.when(done[0] == 0)
        def _step():
            nmc[...] = jnp.full(nmc.shape, BIG, f32)

            def mins(rt):
                sl = pl.ds(rt * T, T)
                adj = mask_ref[sl, :].astype(jnp.int32) == 7
                t_row = jnp.where(adj, lr[sl, :], BIG)
                nmc[...] = jnp.minimum(
                    nmc[...], jnp.min(t_row, axis=0, keepdims=True))
            loop(mins)
            l1c[...] = jnp.minimum(lc[...], nmc[...])

            def jump_c(ct):
                slc = pl.ds(ct * T, T)
                ohc = (l1c[:, slc] == iota_r).astype(f32)
                jc[:, slc] = jax.lax.dot_general(
                    l1c[...], ohc, (((1,), (0,)), ((), ())), precision=hp)
            loop(jump_c)

            new_c = jnp.minimum(l1c[...], jc[...])
            chg = jnp.max((new_c != lc[...]).astype(jnp.int32))
            lc[...] = new_c
            transpose_cols_to_rows(lc, lr)
            done[0] = jnp.where(chg == 0, 1, 0)
        return carry

    jax.lax.fori_loop(0, _ITERS, iter_body, 0)

    corec_b = denc[...] >= _MIN_SAMPLES
    isrc = corec_b.astype(f32) * (lc[...] == iota_c).astype(f32)
    iota_tc = jax.lax.broadcasted_iota(jnp.int32, (1, T), 1)

    def rank_c(ct):
        slc = pl.ds(ct * T, T)
        colg = (iota_tc + ct * T).astype(f32)
        tri = (iota_r <= colg).astype(f32)
        rankc[:, slc] = jax.lax.dot_general(
            isrc, tri, (((1,), (0,)), ((), ())), precision=hp) - 1.0
    loop(rank_c)

    def cc_c(ct):
        slc = pl.ds(ct * T, T)
        ohc = (lc[:, slc] == iota_r).astype(f32)
        ccc[:, slc] = jax.lax.dot_general(
            rankc[...], ohc, (((1,), (0,)), ((), ())), precision=hp)
    loop(cc_c)

    bminc[...] = jnp.full(bminc.shape, BIG, f32)
    any_non_core = jnp.max((~corec_b).astype(jnp.int32))

    @pl.when(any_non_core == 1)
    def _border_phase():
        transpose_cols_to_rows(ccc, ccr)

        def border(rt):
            sl = pl.ds(rt * T, T)
            v = mask_ref[sl, :].astype(jnp.int32)
            cand = jnp.bitwise_and(v, 2) > 0
            t = jnp.where(cand, ccr[sl, :], BIG)
            bminc[...] = jnp.minimum(
                bminc[...], jnp.min(t, axis=0, keepdims=True))
        loop(border)

    borderl = jnp.where(bminc[...] < BIG, bminc[...], -1.0)
    res = jnp.where(corec_b, ccc[...], borderl)
    out_ref[...] = res.astype(jnp.int32)


def kernel(X):
    N, _ = X.shape
    out = pl.pallas_call(
        _dbscan_body,
        out_shape=jax.ShapeDtypeStruct((1, N), jnp.int32),
        scratch_shapes=[
            pltpu.VMEM((N, N), jnp.int8),
            pltpu.VMEM((N, 1), jnp.float32),
            pltpu.VMEM((N, 1), jnp.float32),
            pltpu.VMEM((1, N), jnp.float32),
            pltpu.VMEM((N, 1), jnp.float32),
            pltpu.VMEM((1, N), jnp.float32),
            pltpu.VMEM((1, N), jnp.float32),
            pltpu.VMEM((1, N), jnp.float32),
            pltpu.VMEM((1, N), jnp.float32),
            pltpu.VMEM((1, N), jnp.float32),
            pltpu.VMEM((N, 1), jnp.float32),
            pltpu.VMEM((1, N), jnp.float32),
            pltpu.VMEM((1, N), jnp.float32),
            pltpu.SMEM((1,), jnp.int32),
        ],
        compiler_params=pltpu.CompilerParams(
            vmem_limit_bytes=63 * 1024 * 1024),
    )(X)
    return out.reshape(N)


# speedup vs baseline: 1.0135x; 1.0135x over previous
import jax
import jax.numpy as jnp
from jax.experimental import pallas as pl
from jax.experimental.pallas import tpu as pltpu

_EPS2 = 400.0
_MIN_SAMPLES = 5.0
_ITERS = 24
_T = 256


def _dbscan_body(x_ref, out_ref, mask_ref, sqr, denr, denc, lr, lc,
                 nmc, l1c, jc, rankc, ccr, ccc, bminc, done):
    N = x_ref.shape[0]
    T = _T if N % _T == 0 else N
    NT = N // T
    BIG = jnp.float32(N)
    f32 = jnp.float32
    hp = jax.lax.Precision.HIGHEST

    def loop(body):
        jax.lax.fori_loop(0, NT, lambda i, c: (body(i), 0)[1], 0)

    x = x_ref[...]
    sqr[...] = jnp.sum(x * x, axis=1, keepdims=True)
    denc[...] = jnp.zeros_like(denc[...])
    ones_t = jnp.ones((T, 1), f32)

    def p1(rt):
        sl = pl.ds(rt * T, T)
        xr = x_ref[sl, :]
        mm = jax.lax.dot_general(xr, x, (((1,), (1,)), ((), ())),
                                 precision=jax.lax.Precision.DEFAULT)
        sqc_b = jax.lax.dot_general(ones_t, sqr[...], (((1,), (1,)), ((), ())),
                                    precision=hp)
        d2 = (sqr[sl, :] + sqc_b) - 2.0 * mm
        nb = jnp.maximum(d2, 0.0) <= _EPS2
        mask_ref[sl, :] = nb.astype(jnp.int8)
        nbf = nb.astype(f32)
        denc[...] = denc[...] + jnp.sum(nbf, axis=0, keepdims=True)
        denr[sl, :] = jnp.sum(nbf, axis=1, keepdims=True)
    loop(p1)

    iota_r = jax.lax.broadcasted_iota(jnp.int32, (N, 1), 0).astype(f32)
    iota_c = jax.lax.broadcasted_iota(jnp.int32, (1, N), 1).astype(f32)
    iota_tr = jax.lax.broadcasted_iota(jnp.int32, (T, 1), 0)
    nmc[...] = jnp.full(nmc.shape, BIG, f32)

    def p2(rt):
        sl = pl.ds(rt * T, T)
        m32 = mask_ref[sl, :].astype(jnp.int32)
        corer_t = (denr[sl, :] >= _MIN_SAMPLES).astype(jnp.int32)
        corec_i = (denc[...] >= _MIN_SAMPLES).astype(jnp.int32)
        v = m32 * (1 + 2 * corer_t + 4 * corec_i)
        mask_ref[sl, :] = v.astype(jnp.int8)
        rowg = (iota_tr + rt * T).astype(f32)
        t0 = jnp.where(v == 7, rowg, BIG)
        nmc[...] = jnp.minimum(
            nmc[...], jnp.min(t0, axis=0, keepdims=True))
    loop(p2)

    lc[...] = iota_c
    done[0] = 0

    def transpose_cols_to_rows(src_c, dst_r):
        def tbody(rt):
            sl = pl.ds(rt * T, T)
            rowg = (iota_tr + rt * T).astype(f32)
            eq = (iota_c == rowg).astype(f32)
            dst_r[sl, :] = jax.lax.dot_general(
                eq, src_c[...], (((1,), (1,)), ((), ())), precision=hp)
        loop(tbody)

    def jump_c(ct):
        slc = pl.ds(ct * T, T)
        ohc = (l1c[:, slc] == iota_r).astype(f32)
        jc[:, slc] = jax.lax.dot_general(
            l1c[...], ohc, (((1,), (0,)), ((), ())), precision=hp)

    def prop_tail():
        l1c[...] = jnp.minimum(lc[...], nmc[...])
        loop(jump_c)
        new_c = jnp.minimum(l1c[...], jc[...])
        chg = jnp.max((new_c != lc[...]).astype(jnp.int32))
        lc[...] = new_c
        transpose_cols_to_rows(lc, lr)
        done[0] = jnp.where(chg == 0, 1, 0)

    prop_tail()

    def iter_body(_, carry):
        @pl.when(done[0] == 0)
        def _step():
            nmc[...] = jnp.full(nmc.shape, BIG, f32)

            def mins(rt):
                sl = pl.ds(rt * T, T)
                adj = mask_ref[sl, :].astype(jnp.int32) == 7
                t_row = jnp.where(adj, lr[sl, :], BIG)
                nmc[...] = jnp.minimum(
                    nmc[...], jnp.min(t_row, axis=0, keepdims=True))
            loop(mins)
            prop_tail()
        return carry

    jax.lax.fori_loop(0, _ITERS - 1, iter_body, 0)

    corec_b = denc[...] >= _MIN_SAMPLES
    isrc = corec_b.astype(f32) * (lc[...] == iota_c).astype(f32)
    iota_tc = jax.lax.broadcasted_iota(jnp.int32, (1, T), 1)

    def rank_c(ct):
        slc = pl.ds(ct * T, T)
        colg = (iota_tc + ct * T).astype(f32)
        tri = (iota_r <= colg).astype(f32)
        rankc[:, slc] = jax.lax.dot_general(
            isrc, tri, (((1,), (0,)), ((), ())), precision=hp) - 1.0
    loop(rank_c)

    def cc_c(ct):
        slc = pl.ds(ct * T, T)
        ohc = (lc[:, slc] == iota_r).astype(f32)
        ccc[:, slc] = jax.lax.dot_general(
            rankc[...], ohc, (((1,), (0,)), ((), ())), precision=hp)
    loop(cc_c)

    bminc[...] = jnp.full(bminc.shape, BIG, f32)
    any_non_core = jnp.max((~corec_b).astype(jnp.int32))

    @pl.when(any_non_core == 1)
    def _border_phase():
        transpose_cols_to_rows(ccc, ccr)

        def border(rt):
            sl = pl.ds(rt * T, T)
            v = mask_ref[sl, :].astype(jnp.int32)
            cand = jnp.bitwise_and(v, 2) > 0
            t = jnp.where(cand, ccr[sl, :], BIG)
            bminc[...] = jnp.minimum(
                bminc[...], jnp.min(t, axis=0, keepdims=True))
        loop(border)

    borderl = jnp.where(bminc[...] < BIG, bminc[...], -1.0)
    res = jnp.where(corec_b, ccc[...], borderl)
    out_ref[...] = res.astype(jnp.int32)


def kernel(X):
    N, _ = X.shape
    out = pl.pallas_call(
        _dbscan_body,
        out_shape=jax.ShapeDtypeStruct((1, N), jnp.int32),
        scratch_shapes=[
            pltpu.VMEM((N, N), jnp.int8),
            pltpu.VMEM((N, 1), jnp.float32),
            pltpu.VMEM((N, 1), jnp.float32),
            pltpu.VMEM((1, N), jnp.float32),
            pltpu.VMEM((N, 1), jnp.float32),
            pltpu.VMEM((1, N), jnp.float32),
            pltpu.VMEM((1, N), jnp.float32),
            pltpu.VMEM((1, N), jnp.float32),
            pltpu.VMEM((1, N), jnp.float32),
            pltpu.VMEM((1, N), jnp.float32),
            pltpu.VMEM((N, 1), jnp.float32),
            pltpu.VMEM((1, N), jnp.float32),
            pltpu.VMEM((1, N), jnp.float32),
            pltpu.SMEM((1,), jnp.int32),
        ],
        compiler_params=pltpu.CompilerParams(
            vmem_limit_bytes=63 * 1024 * 1024),
    )(X)
    return out.reshape(N)
